# baseline (speedup 1.0000x reference)
import functools

import jax
import jax.numpy as jnp
from jax import lax
from jax.experimental import pallas as pl
from jax.experimental.pallas import tpu as pltpu
from jax.experimental.pallas import tpu_sc as plsc

N_ITEMS = 1000000
N_CATS = 1000
EMB = 64
CAT_EMB = 16
BATCH = 16384

NC = 2
NS = 16
NW = NC * NS
B_PER_W = BATCH // NW


def _sc_gather(item_ids, cat_ids, item_table, cat_table):
    mesh = plsc.VectorSubcoreMesh(core_axis_name="c", subcore_axis_name="s")

    @functools.partial(
        pl.kernel,
        out_type=(
            jax.ShapeDtypeStruct((BATCH, EMB), jnp.float32),
            jax.ShapeDtypeStruct((BATCH, CAT_EMB), jnp.float32),
        ),
        mesh=mesh,
        scratch_types=[
            pltpu.VMEM((B_PER_W,), jnp.int32),
            pltpu.VMEM((B_PER_W, EMB), jnp.float32),
            pltpu.VMEM((B_PER_W,), jnp.int32),
            pltpu.VMEM((B_PER_W, CAT_EMB), jnp.float32),
            pltpu.SemaphoreType.DMA,
            pltpu.SemaphoreType.DMA,
        ],
        compiler_params=pltpu.CompilerParams(use_tc_tiling_on_sc=False),
    )
    def k(item_ids_hbm, cat_ids_hbm, item_tbl_hbm, cat_tbl_hbm,
          ie_hbm, ce_hbm, idx_v, rows_v, cidx_v, crows_v, sem_i, sem_c):
        wid = lax.axis_index("s") * NC + lax.axis_index("c")
        base = wid * B_PER_W
        pltpu.sync_copy(item_ids_hbm.at[pl.ds(base, B_PER_W)], idx_v)
        item_gather = pltpu.async_copy(item_tbl_hbm.at[idx_v], rows_v, sem_i)
        pltpu.sync_copy(cat_ids_hbm.at[pl.ds(base, B_PER_W)], cidx_v)
        cat_gather = pltpu.async_copy(cat_tbl_hbm.at[cidx_v], crows_v, sem_c)
        item_gather.wait()
        pltpu.sync_copy(rows_v, ie_hbm.at[pl.ds(base, B_PER_W)])
        cat_gather.wait()
        pltpu.sync_copy(crows_v, ce_hbm.at[pl.ds(base, B_PER_W)])

    return k(item_ids, cat_ids, item_table, cat_table)


def _mlp_body(ie_ref, ce_ref, w1a_ref, w1b_ref, b1_ref, w2t_ref, b2_ref,
              out_ref):
    h = jnp.dot(ie_ref[...], w1a_ref[...], preferred_element_type=jnp.float32)
    h += jnp.dot(ce_ref[...], w1b_ref[...], preferred_element_type=jnp.float32)
    h = jnp.maximum(h + b1_ref[...], 0.0)
    out_ref[...] = (
        jnp.dot(h, w2t_ref[...], preferred_element_type=jnp.float32)
        + b2_ref[...]
    )


def _tc_mlp(ie, ce, W1a, W1b, b1, W2T, b2):
    blk = 2048
    grid = (BATCH // blk,)
    return pl.pallas_call(
        _mlp_body,
        grid=grid,
        in_specs=[
            pl.BlockSpec((blk, EMB), lambda i: (i, 0)),
            pl.BlockSpec((blk, CAT_EMB), lambda i: (i, 0)),
            pl.BlockSpec((EMB, EMB), lambda i: (0, 0)),
            pl.BlockSpec((CAT_EMB, EMB), lambda i: (0, 0)),
            pl.BlockSpec((1, EMB), lambda i: (0, 0)),
            pl.BlockSpec((EMB, EMB), lambda i: (0, 0)),
            pl.BlockSpec((1, EMB), lambda i: (0, 0)),
        ],
        out_specs=pl.BlockSpec((blk, EMB), lambda i: (i, 0)),
        out_shape=jax.ShapeDtypeStruct((BATCH, EMB), jnp.float32),
        compiler_params=pltpu.CompilerParams(
            dimension_semantics=("arbitrary",),
        ),
    )(ie, ce, W1a, W1b, b1, W2T, b2)


@jax.jit
def kernel(item_ids, cat_ids, item_table, cat_table, W1, b1, W2, b2):
    ie, ce = _sc_gather(item_ids, cat_ids, item_table, cat_table)
    W1a = W1[:, :EMB].T
    W1b = W1[:, EMB:].T
    W2T = W2.T
    return _tc_mlp(ie, ce, W1a, W1b, b1.reshape(1, EMB), W2T,
                   b2.reshape(1, EMB))
